# baseline (speedup 1.0000x reference)
import functools

import jax
import jax.numpy as jnp
from jax import lax
from jax.experimental import pallas as pl
from jax.experimental.pallas import tpu as pltpu
from jax.experimental.pallas import tpu_sc as plsc

_NC = 2
_NS = 16
_LANES = 16
_CHUNK = 40
_NCHUNKS = 250
_IDXRING = 8
_EDGES_PER_W = _CHUNK * _NCHUNKS


def _sc_segment_sum(N, E, D):
  workers = _NC * _NS
  assert E == workers * _EDGES_PER_W
  slice_rows = 640
  last_rows = N - (_NS - 1) * slice_rows
  mesh = plsc.VectorSubcoreMesh(core_axis_name="c", subcore_axis_name="s")

  @functools.partial(
      pl.kernel,
      out_type=[
          jax.ShapeDtypeStruct((_NC * N, D), jnp.float32),
          jax.ShapeDtypeStruct((_NC * N,), jnp.float32),
      ],
      mesh=mesh,
      scratch_types=[
          pltpu.VMEM((_IDXRING, 2, _CHUNK), jnp.int32),
          pltpu.VMEM((_CHUNK, D), jnp.float32),
          pltpu.VMEM((_CHUNK, D), jnp.float32),
          pltpu.VMEM((_CHUNK, D), jnp.float32),
          pltpu.VMEM((_CHUNK, D), jnp.float32),
          pltpu.VMEM((_CHUNK, D), jnp.float32),
          pltpu.VMEM((_CHUNK, D), jnp.float32),
          pltpu.VMEM((_CHUNK,), jnp.float32),
          pltpu.VMEM((640,), jnp.float32),
          pltpu.VMEM_SHARED((N, D), jnp.float32),
          pltpu.VMEM_SHARED((N,), jnp.float32),
          pltpu.SemaphoreType.DMA,
          pltpu.SemaphoreType.DMA,
          pltpu.SemaphoreType.DMA,
          pltpu.SemaphoreType.DMA,
          pltpu.SemaphoreType.DMA,
          pltpu.SemaphoreType.DMA,
          pltpu.SemaphoreType.DMA,
          pltpu.SemaphoreType.DMA,
      ],
  )
  def k(idx2_hbm, h_hbm, r_hbm, psum_hbm, pcnt_hbm,
        idxr, h0, h1, r0, r1, m0, m1, ones_v, cnt_stage,
        accum_sh, cnt_sh, gs0, gs1, ss0, ss1, cs0, cs1, is0, is1):
    c = lax.axis_index("c")
    s = lax.axis_index("s")
    w = s * _NC + c
    hrow = (h0, h1)
    rrow = (r0, r1)
    mrow = (m0, m1)
    gsem = (gs0, gs1)
    ssem = (ss0, ss1)
    csem = (cs0, cs1)
    isem = (is0, is1)
    ebase = pl.multiple_of(w * _EDGES_PER_W, 8)

    zero16 = jnp.zeros((_LANES,), jnp.float32)
    one16 = jnp.ones((_LANES,), jnp.float32)

    def zrow(j, carry):
      for t in range(D // _LANES):
        m0[j, pl.ds(t * _LANES, _LANES)] = zero16
      return carry
    lax.fori_loop(0, _CHUNK, zrow, 0)
    for t in range(640 // _LANES):
      cnt_stage[pl.ds(t * _LANES, _LANES)] = zero16
    for t in range(_CHUNK // _LANES):
      ones_v[pl.ds(t * _LANES, _LANES)] = one16
    ones_v[pl.ds(_CHUNK - _LANES, _LANES)] = one16

    base_row = pl.multiple_of(s * slice_rows, 8)

    def _zero_rows(nrows):
      pltpu.sync_copy(cnt_stage.at[pl.ds(0, nrows)],
                      cnt_sh.at[pl.ds(base_row, nrows)])
      for q in range(nrows // _CHUNK):
        ro = pl.multiple_of(base_row + q * _CHUNK, 8)
        pltpu.sync_copy(m0, accum_sh.at[pl.ds(ro, _CHUNK), :])

    @pl.when(s < _NS - 1)
    def _():
      _zero_rows(slice_rows)

    @pl.when(s == _NS - 1)
    def _():
      _zero_rows(last_rows)

    plsc.subcore_barrier()

    def _fetch(j, b, slot):
      e0 = pl.multiple_of(ebase + j * _CHUNK, 8)
      pltpu.async_copy(h_hbm.at[idxr.at[slot, 0]], hrow[b], gsem[b])
      pltpu.async_copy(r_hbm.at[pl.ds(e0, _CHUNK), :], rrow[b], gsem[b])

    def _wait_fetch(b):
      pltpu.make_async_copy(h_hbm.at[idxr.at[0, 0]], hrow[b], gsem[b]).wait()
      pltpu.make_async_copy(r_hbm.at[pl.ds(0, _CHUNK), :], rrow[b],
                            gsem[b]).wait()

    def _wait_scatter(b):
      pltpu.make_async_copy(mrow[b], accum_sh.at[idxr.at[0, 1]],
                            ssem[b]).wait()
      pltpu.make_async_copy(ones_v, cnt_sh.at[idxr.at[0, 1]],
                            csem[b]).wait()

    def _fetch_idx(j, sem):
      slot = lax.rem(j, _IDXRING)
      pltpu.async_copy(idx2_hbm.at[w, j], idxr.at[slot], sem)

    def _wait_idx(sem):
      pltpu.make_async_copy(idx2_hbm.at[w, 0], idxr.at[0], sem).wait()

    def _chunk(j, b):
      bn = 1 - b
      _wait_fetch(b)
      @pl.when(j >= 2)
      def _():
        _wait_scatter(b)

      @plsc.parallel_loop(0, _CHUNK, 1, unroll=2)
      def _(j2):
        for t in range(D // _LANES):
          sl = pl.ds(t * _LANES, _LANES)
          mrow[b][j2, sl] = hrow[b][j2, sl] + rrow[b][j2, sl]

      slot_j = lax.rem(j, _IDXRING)
      pltpu.async_copy(mrow[b], accum_sh.at[idxr.at[slot_j, 1]], ssem[b],
                       add=True)
      pltpu.async_copy(ones_v, cnt_sh.at[idxr.at[slot_j, 1]], csem[b],
                       add=True)

      @pl.when(j + 2 < _NCHUNKS)
      def _():
        _wait_idx(isem[b])
        _fetch(j + 2, b, lax.rem(j + 2, _IDXRING))

      @pl.when(j + 3 < _NCHUNKS)
      def _():
        _fetch_idx(j + 3, isem[bn])

    pltpu.sync_copy(idx2_hbm.at[w, 0], idxr.at[0])
    pltpu.sync_copy(idx2_hbm.at[w, 1], idxr.at[1])
    _fetch_idx(2, isem[0])
    _fetch(0, 0, 0)
    _fetch(1, 1, 1)

    def body(p, carry):
      j = p * 2
      _chunk(j, 0)
      _chunk(j + 1, 1)
      return carry
    lax.fori_loop(0, _NCHUNKS // 2, body, 0)

    _wait_scatter(0)
    _wait_scatter(1)

    plsc.subcore_barrier()

    def _writeback(nrows):
      ro = base_row
      out_ro = pl.multiple_of(c * N + base_row, 8)
      pltpu.sync_copy(accum_sh.at[pl.ds(ro, nrows), :],
                      psum_hbm.at[pl.ds(out_ro, nrows), :])
      pltpu.sync_copy(cnt_sh.at[pl.ds(ro, nrows)],
                      cnt_stage.at[pl.ds(0, nrows)])
      pltpu.sync_copy(cnt_stage.at[pl.ds(0, nrows)],
                      pcnt_hbm.at[pl.ds(out_ro, nrows)])

    @pl.when(s < _NS - 1)
    def _():
      _writeback(slice_rows)

    @pl.when(s == _NS - 1)
    def _():
      _writeback(last_rows)

  return k


def _tc_finish(N, D):
  blk = 1000
  def body(ps_ref, pc_ref, w_ref, b_ref, o_ref):
    ssum = ps_ref[0] + ps_ref[1]
    cnt = (pc_ref[0, 0, 0] + pc_ref[1, 0, 0]).reshape(blk, 1)
    feat = ssum / jnp.maximum(cnt, 1.0)
    o_ref[...] = jnp.dot(feat, w_ref[...],
                         preferred_element_type=jnp.float32) + b_ref[...]
  return pl.pallas_call(
      body,
      grid=(N // blk,),
      in_specs=[
          pl.BlockSpec((_NC, blk, D), lambda i: (0, i, 0)),
          pl.BlockSpec((_NC, 1, 1, blk), lambda i: (0, i, 0, 0)),
          pl.BlockSpec((D, D), lambda i: (0, 0)),
          pl.BlockSpec((1, D), lambda i: (0, 0)),
      ],
      out_specs=pl.BlockSpec((blk, D), lambda i: (i, 0)),
      out_shape=jax.ShapeDtypeStruct((N, D), jnp.float32),
  )


def kernel(h, r, edge_index, W, b):
  N, D = h.shape
  E = r.shape[0]
  workers = _NC * _NS
  idx2 = jnp.stack(
      [edge_index[0].reshape(workers, _NCHUNKS, _CHUNK),
       edge_index[1].reshape(workers, _NCHUNKS, _CHUNK)], axis=2)
  psum, pcnt = _sc_segment_sum(N, E, D)(idx2, h, r)
  psum = psum.reshape(_NC, N, D)
  pcnt = pcnt.reshape(_NC, N // 1000, 1, 1000)
  return _tc_finish(N, D)(psum, pcnt, W, b.reshape(1, D))


# speedup vs baseline: 1.0115x; 1.0115x over previous
import functools

import jax
import jax.numpy as jnp
from jax import lax
from jax.experimental import pallas as pl
from jax.experimental.pallas import tpu as pltpu
from jax.experimental.pallas import tpu_sc as plsc

_NC = 2
_NS = 16
_LANES = 16
_CHUNK = 40
_NCHUNKS = 250
_IDXRING = 8
_EDGES_PER_W = _CHUNK * _NCHUNKS


def _sc_segment_sum(N, E, D):
  workers = _NC * _NS
  assert E == workers * _EDGES_PER_W
  slice_rows = 640
  last_rows = N - (_NS - 1) * slice_rows
  mesh = plsc.VectorSubcoreMesh(core_axis_name="c", subcore_axis_name="s")

  @functools.partial(
      pl.kernel,
      out_type=[
          jax.ShapeDtypeStruct((_NC * N, D), jnp.float32),
          jax.ShapeDtypeStruct((_NC * N,), jnp.float32),
      ],
      mesh=mesh,
      scratch_types=[
          pltpu.VMEM((_IDXRING, 2, _CHUNK), jnp.int32),
          pltpu.VMEM((_CHUNK, D), jnp.float32),
          pltpu.VMEM((_CHUNK, D), jnp.float32),
          pltpu.VMEM((_CHUNK, D), jnp.float32),
          pltpu.VMEM((_CHUNK, D), jnp.float32),
          pltpu.VMEM((_CHUNK, D), jnp.float32),
          pltpu.VMEM((_CHUNK, D), jnp.float32),
          pltpu.VMEM((_CHUNK,), jnp.float32),
          pltpu.VMEM((640,), jnp.float32),
          pltpu.VMEM_SHARED((N, D), jnp.float32),
          pltpu.VMEM_SHARED((N,), jnp.float32),
          pltpu.SemaphoreType.DMA,
          pltpu.SemaphoreType.DMA,
          pltpu.SemaphoreType.DMA,
          pltpu.SemaphoreType.DMA,
          pltpu.SemaphoreType.DMA,
          pltpu.SemaphoreType.DMA,
          pltpu.SemaphoreType.DMA,
          pltpu.SemaphoreType.DMA,
      ],
  )
  def k(idx2_hbm, h_hbm, r_hbm, psum_hbm, pcnt_hbm,
        idxr, h0, h1, r0, r1, m0, m1, ones_v, cnt_stage,
        accum_sh, cnt_sh, gs0, gs1, ss0, ss1, cs0, cs1, is0, is1):
    c = lax.axis_index("c")
    s = lax.axis_index("s")
    w = s * _NC + c
    hrow = (h0, h1)
    rrow = (r0, r1)
    mrow = (m0, m1)
    gsem = (gs0, gs1)
    ssem = (ss0, ss1)
    csem = (cs0, cs1)
    isem = (is0, is1)
    ebase = pl.multiple_of(w * _EDGES_PER_W, 8)

    zero16 = jnp.zeros((_LANES,), jnp.float32)
    one16 = jnp.ones((_LANES,), jnp.float32)

    def zrow(j, carry):
      for t in range(D // _LANES):
        m0[j, pl.ds(t * _LANES, _LANES)] = zero16
      return carry
    lax.fori_loop(0, _CHUNK, zrow, 0)
    for t in range(640 // _LANES):
      cnt_stage[pl.ds(t * _LANES, _LANES)] = zero16
    for t in range(_CHUNK // _LANES):
      ones_v[pl.ds(t * _LANES, _LANES)] = one16
    ones_v[pl.ds(_CHUNK - _LANES, _LANES)] = one16

    base_row = pl.multiple_of(s * slice_rows, 8)

    def _zero_rows(nrows):
      pltpu.sync_copy(cnt_stage.at[pl.ds(0, nrows)],
                      cnt_sh.at[pl.ds(base_row, nrows)])
      for q in range(nrows // _CHUNK):
        ro = pl.multiple_of(base_row + q * _CHUNK, 8)
        pltpu.sync_copy(m0, accum_sh.at[pl.ds(ro, _CHUNK), :])

    @pl.when(s < _NS - 1)
    def _():
      _zero_rows(slice_rows)

    @pl.when(s == _NS - 1)
    def _():
      _zero_rows(last_rows)

    plsc.subcore_barrier()

    def _fetch(j, b, slot):
      e0 = pl.multiple_of(ebase + j * _CHUNK, 8)
      pltpu.async_copy(h_hbm.at[idxr.at[slot, 0]], hrow[b], gsem[b])
      pltpu.async_copy(r_hbm.at[pl.ds(e0, _CHUNK), :], rrow[b], gsem[b])

    def _wait_fetch(b):
      pltpu.make_async_copy(h_hbm.at[idxr.at[0, 0]], hrow[b], gsem[b]).wait()
      pltpu.make_async_copy(r_hbm.at[pl.ds(0, _CHUNK), :], rrow[b],
                            gsem[b]).wait()

    def _wait_scatter(b):
      pltpu.make_async_copy(mrow[b], accum_sh.at[idxr.at[0, 1]],
                            ssem[b]).wait()
      pltpu.make_async_copy(ones_v, cnt_sh.at[idxr.at[0, 1]],
                            csem[b]).wait()

    def _fetch_idx(j, sem):
      slot = lax.rem(j, _IDXRING)
      pltpu.async_copy(idx2_hbm.at[w, j], idxr.at[slot], sem)

    def _wait_idx(sem):
      pltpu.make_async_copy(idx2_hbm.at[w, 0], idxr.at[0], sem).wait()

    def _chunk(j, b):
      bn = 1 - b
      _wait_fetch(b)
      @pl.when(j >= 2)
      def _():
        _wait_scatter(b)

      def addrow(j2, carry2):
        for t in range(D // _LANES):
          sl = pl.ds(t * _LANES, _LANES)
          mrow[b][j2, sl] = hrow[b][j2, sl] + rrow[b][j2, sl]
        return carry2
      lax.fori_loop(0, _CHUNK, addrow, 0)

      slot_j = lax.rem(j, _IDXRING)
      pltpu.async_copy(mrow[b], accum_sh.at[idxr.at[slot_j, 1]], ssem[b],
                       add=True)
      pltpu.async_copy(ones_v, cnt_sh.at[idxr.at[slot_j, 1]], csem[b],
                       add=True)

      @pl.when(j + 2 < _NCHUNKS)
      def _():
        _wait_idx(isem[b])
        _fetch(j + 2, b, lax.rem(j + 2, _IDXRING))

      @pl.when(j + 3 < _NCHUNKS)
      def _():
        _fetch_idx(j + 3, isem[bn])

    pltpu.sync_copy(idx2_hbm.at[w, 0], idxr.at[0])
    pltpu.sync_copy(idx2_hbm.at[w, 1], idxr.at[1])
    _fetch_idx(2, isem[0])
    _fetch(0, 0, 0)
    _fetch(1, 1, 1)

    def body(p, carry):
      j = p * 2
      _chunk(j, 0)
      _chunk(j + 1, 1)
      return carry
    lax.fori_loop(0, _NCHUNKS // 2, body, 0)

    _wait_scatter(0)
    _wait_scatter(1)

    plsc.subcore_barrier()

    def _writeback(nrows):
      ro = base_row
      out_ro = pl.multiple_of(c * N + base_row, 8)
      pltpu.sync_copy(accum_sh.at[pl.ds(ro, nrows), :],
                      psum_hbm.at[pl.ds(out_ro, nrows), :])
      pltpu.sync_copy(cnt_sh.at[pl.ds(ro, nrows)],
                      cnt_stage.at[pl.ds(0, nrows)])
      pltpu.sync_copy(cnt_stage.at[pl.ds(0, nrows)],
                      pcnt_hbm.at[pl.ds(out_ro, nrows)])

    @pl.when(s < _NS - 1)
    def _():
      _writeback(slice_rows)

    @pl.when(s == _NS - 1)
    def _():
      _writeback(last_rows)

  return k


def _tc_finish(N, D):
  blk = 1000
  def body(ps_ref, pc_ref, w_ref, b_ref, o_ref):
    ssum = ps_ref[0] + ps_ref[1]
    cnt = (pc_ref[0, 0, 0] + pc_ref[1, 0, 0]).reshape(blk, 1)
    feat = ssum / jnp.maximum(cnt, 1.0)
    o_ref[...] = jnp.dot(feat, w_ref[...],
                         preferred_element_type=jnp.float32) + b_ref[...]
  return pl.pallas_call(
      body,
      grid=(N // blk,),
      in_specs=[
          pl.BlockSpec((_NC, blk, D), lambda i: (0, i, 0)),
          pl.BlockSpec((_NC, 1, 1, blk), lambda i: (0, i, 0, 0)),
          pl.BlockSpec((D, D), lambda i: (0, 0)),
          pl.BlockSpec((1, D), lambda i: (0, 0)),
      ],
      out_specs=pl.BlockSpec((blk, D), lambda i: (i, 0)),
      out_shape=jax.ShapeDtypeStruct((N, D), jnp.float32),
  )


def kernel(h, r, edge_index, W, b):
  N, D = h.shape
  E = r.shape[0]
  workers = _NC * _NS
  idx2 = jnp.stack(
      [edge_index[0].reshape(workers, _NCHUNKS, _CHUNK),
       edge_index[1].reshape(workers, _NCHUNKS, _CHUNK)], axis=2)
  psum, pcnt = _sc_segment_sum(N, E, D)(idx2, h, r)
  psum = psum.reshape(_NC, N, D)
  pcnt = pcnt.reshape(_NC, N // 1000, 1, 1000)
  return _tc_finish(N, D)(psum, pcnt, W, b.reshape(1, D))


# speedup vs baseline: 1.1046x; 1.0920x over previous
import functools

import jax
import jax.numpy as jnp
from jax import lax
from jax.experimental import pallas as pl
from jax.experimental.pallas import tpu as pltpu
from jax.experimental.pallas import tpu_sc as plsc

_NC = 2
_NS = 16
_LANES = 16
_CHUNK = 40
_NCHUNKS = 250
_IDXRING = 8
_EDGES_PER_W = _CHUNK * _NCHUNKS


def _sc_segment_sum(N, E, D):
  workers = _NC * _NS
  assert E == workers * _EDGES_PER_W
  slice_rows = 640
  last_rows = N - (_NS - 1) * slice_rows
  mesh = plsc.VectorSubcoreMesh(core_axis_name="c", subcore_axis_name="s")

  @functools.partial(
      pl.kernel,
      out_type=[
          jax.ShapeDtypeStruct((_NC * N, D), jnp.float32),
          jax.ShapeDtypeStruct((_NC * N,), jnp.float32),
      ],
      mesh=mesh,
      scratch_types=[
          pltpu.VMEM((_IDXRING, _CHUNK), jnp.int32),
          pltpu.VMEM((_IDXRING, _CHUNK), jnp.int32),
          pltpu.VMEM((_CHUNK, D), jnp.float32),
          pltpu.VMEM((_CHUNK, D), jnp.float32),
          pltpu.VMEM((_CHUNK, D), jnp.float32),
          pltpu.VMEM((_CHUNK, D), jnp.float32),
          pltpu.VMEM((_CHUNK, D), jnp.float32),
          pltpu.VMEM((_CHUNK, D), jnp.float32),
          pltpu.VMEM((_CHUNK,), jnp.float32),
          pltpu.VMEM((640,), jnp.float32),
          pltpu.VMEM_SHARED((N, D), jnp.float32),
          pltpu.VMEM_SHARED((N,), jnp.float32),
          pltpu.SemaphoreType.DMA,
          pltpu.SemaphoreType.DMA,
          pltpu.SemaphoreType.DMA,
          pltpu.SemaphoreType.DMA,
          pltpu.SemaphoreType.DMA,
          pltpu.SemaphoreType.DMA,
          pltpu.SemaphoreType.DMA,
          pltpu.SemaphoreType.DMA,
      ],
  )
  def k(src_hbm, dst_hbm, h_hbm, r_hbm, psum_hbm, pcnt_hbm,
        idx_s, idx_d, h0, h1, r0, r1, m0, m1, ones_v, cnt_stage,
        accum_sh, cnt_sh, gs0, gs1, ss0, ss1, cs0, cs1, is0, is1):
    c = lax.axis_index("c")
    s = lax.axis_index("s")
    w = s * _NC + c
    hrow = (h0, h1)
    rrow = (r0, r1)
    mrow = (m0, m1)
    gsem = (gs0, gs1)
    ssem = (ss0, ss1)
    csem = (cs0, cs1)
    isem = (is0, is1)
    ebase = pl.multiple_of(w * _EDGES_PER_W, 8)

    zero16 = jnp.zeros((_LANES,), jnp.float32)
    one16 = jnp.ones((_LANES,), jnp.float32)

    def zrow(j, carry):
      for t in range(D // _LANES):
        m0[j, pl.ds(t * _LANES, _LANES)] = zero16
      return carry
    lax.fori_loop(0, _CHUNK, zrow, 0)
    for t in range(640 // _LANES):
      cnt_stage[pl.ds(t * _LANES, _LANES)] = zero16
    for t in range(_CHUNK // _LANES):
      ones_v[pl.ds(t * _LANES, _LANES)] = one16
    ones_v[pl.ds(_CHUNK - _LANES, _LANES)] = one16

    base_row = pl.multiple_of(s * slice_rows, 8)

    def _zero_rows(nrows):
      pltpu.sync_copy(cnt_stage.at[pl.ds(0, nrows)],
                      cnt_sh.at[pl.ds(base_row, nrows)])
      for q in range(nrows // _CHUNK):
        ro = pl.multiple_of(base_row + q * _CHUNK, 8)
        pltpu.sync_copy(m0, accum_sh.at[pl.ds(ro, _CHUNK), :])

    @pl.when(s < _NS - 1)
    def _():
      _zero_rows(slice_rows)

    @pl.when(s == _NS - 1)
    def _():
      _zero_rows(last_rows)

    plsc.subcore_barrier()

    def _fetch(j, b, slot):
      e0 = pl.multiple_of(ebase + j * _CHUNK, 8)
      pltpu.async_copy(h_hbm.at[idx_s.at[slot]], hrow[b], gsem[b])
      pltpu.async_copy(r_hbm.at[pl.ds(e0, _CHUNK), :], rrow[b], gsem[b])

    def _wait_fetch(b):
      pltpu.make_async_copy(h_hbm.at[idx_s.at[0]], hrow[b], gsem[b]).wait()
      pltpu.make_async_copy(r_hbm.at[pl.ds(0, _CHUNK), :], rrow[b],
                            gsem[b]).wait()

    def _wait_scatter(b):
      pltpu.make_async_copy(mrow[b], accum_sh.at[idx_d.at[0]],
                            ssem[b]).wait()
      pltpu.make_async_copy(ones_v, cnt_sh.at[idx_d.at[0]], csem[b]).wait()

    def _fetch_idx(j, sem):
      slot = lax.rem(j, _IDXRING)
      pltpu.async_copy(src_hbm.at[w, j], idx_s.at[slot], sem)
      pltpu.async_copy(dst_hbm.at[w, j], idx_d.at[slot], sem)

    def _wait_idx(sem):
      pltpu.make_async_copy(src_hbm.at[w, 0], idx_s.at[0], sem).wait()
      pltpu.make_async_copy(dst_hbm.at[w, 0], idx_d.at[0], sem).wait()

    def _chunk(j, b):
      bn = 1 - b
      _wait_fetch(b)
      @pl.when(j >= 2)
      def _():
        _wait_scatter(b)

      def addrow(j2, carry2):
        for t in range(D // _LANES):
          sl = pl.ds(t * _LANES, _LANES)
          mrow[b][j2, sl] = hrow[b][j2, sl] + rrow[b][j2, sl]
        return carry2
      lax.fori_loop(0, _CHUNK, addrow, 0)

      slot_j = lax.rem(j, _IDXRING)
      pltpu.async_copy(mrow[b], accum_sh.at[idx_d.at[slot_j]], ssem[b],
                       add=True)
      pltpu.async_copy(ones_v, cnt_sh.at[idx_d.at[slot_j]], csem[b],
                       add=True)

      @pl.when(j + 2 < _NCHUNKS)
      def _():
        _wait_idx(isem[b])
        _fetch(j + 2, b, lax.rem(j + 2, _IDXRING))

      @pl.when(j + 3 < _NCHUNKS)
      def _():
        _fetch_idx(j + 3, isem[bn])

    pltpu.sync_copy(src_hbm.at[w, 0], idx_s.at[0])
    pltpu.sync_copy(dst_hbm.at[w, 0], idx_d.at[0])
    pltpu.sync_copy(src_hbm.at[w, 1], idx_s.at[1])
    pltpu.sync_copy(dst_hbm.at[w, 1], idx_d.at[1])
    _fetch_idx(2, isem[0])
    _fetch(0, 0, 0)
    _fetch(1, 1, 1)

    def body(p, carry):
      j = p * 2
      _chunk(j, 0)
      _chunk(j + 1, 1)
      return carry
    lax.fori_loop(0, _NCHUNKS // 2, body, 0)

    _wait_scatter(0)
    _wait_scatter(1)

    plsc.subcore_barrier()

    def _writeback(nrows):
      ro = base_row
      out_ro = pl.multiple_of(c * N + base_row, 8)
      pltpu.sync_copy(accum_sh.at[pl.ds(ro, nrows), :],
                      psum_hbm.at[pl.ds(out_ro, nrows), :])
      pltpu.sync_copy(cnt_sh.at[pl.ds(ro, nrows)],
                      cnt_stage.at[pl.ds(0, nrows)])
      pltpu.sync_copy(cnt_stage.at[pl.ds(0, nrows)],
                      pcnt_hbm.at[pl.ds(out_ro, nrows)])

    @pl.when(s < _NS - 1)
    def _():
      _writeback(slice_rows)

    @pl.when(s == _NS - 1)
    def _():
      _writeback(last_rows)

  return k


def _tc_finish(N, D):
  blk = 1000
  def body(ps_ref, pc_ref, w_ref, b_ref, o_ref):
    ssum = ps_ref[0] + ps_ref[1]
    cnt = (pc_ref[0, 0, 0] + pc_ref[1, 0, 0]).reshape(blk, 1)
    feat = ssum / jnp.maximum(cnt, 1.0)
    o_ref[...] = jnp.dot(feat, w_ref[...],
                         preferred_element_type=jnp.float32) + b_ref[...]
  return pl.pallas_call(
      body,
      grid=(N // blk,),
      in_specs=[
          pl.BlockSpec((_NC, blk, D), lambda i: (0, i, 0)),
          pl.BlockSpec((_NC, 1, 1, blk), lambda i: (0, i, 0, 0)),
          pl.BlockSpec((D, D), lambda i: (0, 0)),
          pl.BlockSpec((1, D), lambda i: (0, 0)),
      ],
      out_specs=pl.BlockSpec((blk, D), lambda i: (i, 0)),
      out_shape=jax.ShapeDtypeStruct((N, D), jnp.float32),
  )


def kernel(h, r, edge_index, W, b):
  N, D = h.shape
  E = r.shape[0]
  workers = _NC * _NS
  src = edge_index[0].reshape(workers, _NCHUNKS, _CHUNK)
  dst = edge_index[1].reshape(workers, _NCHUNKS, _CHUNK)
  psum, pcnt = _sc_segment_sum(N, E, D)(src, dst, h, r)
  psum = psum.reshape(_NC, N, D)
  pcnt = pcnt.reshape(_NC, N // 1000, 1, 1000)
  return _tc_finish(N, D)(psum, pcnt, W, b.reshape(1, D))


# speedup vs baseline: 1.1370x; 1.0293x over previous
import functools

import jax
import jax.numpy as jnp
from jax import lax
from jax.experimental import pallas as pl
from jax.experimental.pallas import tpu as pltpu
from jax.experimental.pallas import tpu_sc as plsc

_NC = 2
_NS = 16
_LANES = 16
_CHUNK = 40
_NCHUNKS = 250
_BLKCH = 10
_NBLK = _NCHUNKS // _BLKCH
_SLABRING = 3
_EDGES_PER_W = _CHUNK * _NCHUNKS


def _sc_segment_sum(N, E, D):
  workers = _NC * _NS
  assert E == workers * _EDGES_PER_W
  slice_rows = 640
  last_rows = N - (_NS - 1) * slice_rows
  mesh = plsc.VectorSubcoreMesh(core_axis_name="c", subcore_axis_name="s")

  @functools.partial(
      pl.kernel,
      out_type=[
          jax.ShapeDtypeStruct((_NC * N, D), jnp.float32),
          jax.ShapeDtypeStruct((_NC * N,), jnp.float32),
      ],
      mesh=mesh,
      scratch_types=[
          pltpu.VMEM((_SLABRING, 2, _BLKCH, _CHUNK), jnp.int32),
          pltpu.VMEM((_CHUNK, D), jnp.float32),
          pltpu.VMEM((_CHUNK, D), jnp.float32),
          pltpu.VMEM((_CHUNK, D), jnp.float32),
          pltpu.VMEM((_CHUNK, D), jnp.float32),
          pltpu.VMEM((_CHUNK, D), jnp.float32),
          pltpu.VMEM((_CHUNK, D), jnp.float32),
          pltpu.VMEM((_CHUNK,), jnp.float32),
          pltpu.VMEM((640,), jnp.float32),
          pltpu.VMEM_SHARED((N, D), jnp.float32),
          pltpu.VMEM_SHARED((N,), jnp.float32),
          pltpu.SemaphoreType.DMA,
          pltpu.SemaphoreType.DMA,
          pltpu.SemaphoreType.DMA,
          pltpu.SemaphoreType.DMA,
          pltpu.SemaphoreType.DMA,
          pltpu.SemaphoreType.DMA,
          pltpu.SemaphoreType.DMA,
          pltpu.SemaphoreType.DMA,
          pltpu.SemaphoreType.DMA,
      ],
  )
  def k(eidx_hbm, h_hbm, r_hbm, psum_hbm, pcnt_hbm,
        idxr, h0, h1, r0, r1, m0, m1, ones_v, cnt_stage,
        accum_sh, cnt_sh, gs0, gs1, ss0, ss1, cs0, cs1, is0, is1, is2):
    c = lax.axis_index("c")
    s = lax.axis_index("s")
    w = s * _NC + c
    hrow = (h0, h1)
    rrow = (r0, r1)
    mrow = (m0, m1)
    gsem = (gs0, gs1)
    ssem = (ss0, ss1)
    csem = (cs0, cs1)
    isem = (is0, is1, is2)
    ebase = pl.multiple_of(w * _EDGES_PER_W, 8)

    zero16 = jnp.zeros((_LANES,), jnp.float32)
    one16 = jnp.ones((_LANES,), jnp.float32)

    def zrow(j, carry):
      for t in range(D // _LANES):
        m0[j, pl.ds(t * _LANES, _LANES)] = zero16
      return carry
    lax.fori_loop(0, _CHUNK, zrow, 0)
    for t in range(640 // _LANES):
      cnt_stage[pl.ds(t * _LANES, _LANES)] = zero16
    for t in range(_CHUNK // _LANES):
      ones_v[pl.ds(t * _LANES, _LANES)] = one16
    ones_v[pl.ds(_CHUNK - _LANES, _LANES)] = one16

    base_row = pl.multiple_of(s * slice_rows, 8)

    def _zero_rows(nrows):
      pltpu.sync_copy(cnt_stage.at[pl.ds(0, nrows)],
                      cnt_sh.at[pl.ds(base_row, nrows)])
      for q in range(nrows // _CHUNK):
        ro = pl.multiple_of(base_row + q * _CHUNK, 8)
        pltpu.sync_copy(m0, accum_sh.at[pl.ds(ro, _CHUNK), :])

    @pl.when(s < _NS - 1)
    def _():
      _zero_rows(slice_rows)

    @pl.when(s == _NS - 1)
    def _():
      _zero_rows(last_rows)

    plsc.subcore_barrier()

    def _fetch(j, b, fslot, frow):
      e0 = pl.multiple_of(ebase + j * _CHUNK, 8)
      pltpu.async_copy(h_hbm.at[idxr.at[fslot, 0, frow]], hrow[b], gsem[b])
      pltpu.async_copy(r_hbm.at[pl.ds(e0, _CHUNK), :], rrow[b], gsem[b])

    def _wait_fetch(b):
      pltpu.make_async_copy(h_hbm.at[idxr.at[0, 0, 0]], hrow[b],
                            gsem[b]).wait()
      pltpu.make_async_copy(r_hbm.at[pl.ds(0, _CHUNK), :], rrow[b],
                            gsem[b]).wait()

    def _wait_scatter(b):
      pltpu.make_async_copy(mrow[b], accum_sh.at[idxr.at[0, 1, 0]],
                            ssem[b]).wait()
      pltpu.make_async_copy(ones_v, cnt_sh.at[idxr.at[0, 1, 0]],
                            csem[b]).wait()

    def _stage_slab(blk, slot):
      pltpu.async_copy(eidx_hbm.at[0, w, blk], idxr.at[slot, 0], isem[slot])
      pltpu.async_copy(eidx_hbm.at[1, w, blk], idxr.at[slot, 1], isem[slot])

    def _wait_slab(slot):
      pltpu.make_async_copy(eidx_hbm.at[0, w, 0], idxr.at[0, 0],
                            isem[slot]).wait()
      pltpu.make_async_copy(eidx_hbm.at[1, w, 0], idxr.at[0, 1],
                            isem[slot]).wait()

    def _chunk(j, b, slot, row, fslot, frow):
      _wait_fetch(b)
      @pl.when(j >= 2)
      def _():
        _wait_scatter(b)

      def addrow(j2, carry2):
        for t in range(D // _LANES):
          sl = pl.ds(t * _LANES, _LANES)
          mrow[b][j2, sl] = hrow[b][j2, sl] + rrow[b][j2, sl]
        return carry2
      lax.fori_loop(0, _CHUNK, addrow, 0)

      pltpu.async_copy(mrow[b], accum_sh.at[idxr.at[slot, 1, row]], ssem[b],
                       add=True)
      pltpu.async_copy(ones_v, cnt_sh.at[idxr.at[slot, 1, row]], csem[b],
                       add=True)

      @pl.when(j + 2 < _NCHUNKS)
      def _():
        _fetch(j + 2, b, fslot, frow)

    pltpu.sync_copy(eidx_hbm.at[0, w, 0], idxr.at[0, 0])
    pltpu.sync_copy(eidx_hbm.at[1, w, 0], idxr.at[0, 1])
    _fetch(0, 0, 0, 0)
    _fetch(1, 1, 0, 1)

    def _block(blk, slot, last):
      nslot = (slot + 1) % _SLABRING
      jbase = blk * _BLKCH
      if not last:
        _stage_slab(blk + 1, nslot)

      def pair_body(p, carry2):
        row = p * 2
        j = jbase + row
        _chunk(j, 0, slot, row, slot, row + 2)
        _chunk(j + 1, 1, slot, row + 1, slot, row + 3)
        return carry2
      lax.fori_loop(0, _BLKCH // 2 - 1, pair_body, 0)

      if not last:
        _wait_slab(nslot)
      _chunk(jbase + _BLKCH - 2, 0, slot, _BLKCH - 2, nslot, 0)
      _chunk(jbase + _BLKCH - 1, 1, slot, _BLKCH - 1, nslot, 1)

    def sblk_body(q, carry):
      blk = q * _SLABRING
      for o in range(_SLABRING):
        _block(blk + o, o, False)
      return carry
    lax.fori_loop(0, (_NBLK - 1) // _SLABRING, sblk_body, 0)
    _block(_NBLK - 1, (_NBLK - 1) % _SLABRING, True)

    _wait_scatter(0)
    _wait_scatter(1)

    plsc.subcore_barrier()

    def _writeback(nrows):
      ro = base_row
      out_ro = pl.multiple_of(c * N + base_row, 8)
      pltpu.sync_copy(accum_sh.at[pl.ds(ro, nrows), :],
                      psum_hbm.at[pl.ds(out_ro, nrows), :])
      pltpu.sync_copy(cnt_sh.at[pl.ds(ro, nrows)],
                      cnt_stage.at[pl.ds(0, nrows)])
      pltpu.sync_copy(cnt_stage.at[pl.ds(0, nrows)],
                      pcnt_hbm.at[pl.ds(out_ro, nrows)])

    @pl.when(s < _NS - 1)
    def _():
      _writeback(slice_rows)

    @pl.when(s == _NS - 1)
    def _():
      _writeback(last_rows)

  return k


def _tc_finish(N, D):
  blk = 1000
  def body(ps_ref, pc_ref, w_ref, b_ref, o_ref):
    ssum = ps_ref[0] + ps_ref[1]
    cnt = (pc_ref[0, 0, 0] + pc_ref[1, 0, 0]).reshape(blk, 1)
    feat = ssum / jnp.maximum(cnt, 1.0)
    o_ref[...] = jnp.dot(feat, w_ref[...],
                         preferred_element_type=jnp.float32) + b_ref[...]
  return pl.pallas_call(
      body,
      grid=(N // blk,),
      in_specs=[
          pl.BlockSpec((_NC, blk, D), lambda i: (0, i, 0)),
          pl.BlockSpec((_NC, 1, 1, blk), lambda i: (0, i, 0, 0)),
          pl.BlockSpec((D, D), lambda i: (0, 0)),
          pl.BlockSpec((1, D), lambda i: (0, 0)),
      ],
      out_specs=pl.BlockSpec((blk, D), lambda i: (i, 0)),
      out_shape=jax.ShapeDtypeStruct((N, D), jnp.float32),
  )


def kernel(h, r, edge_index, W, b):
  N, D = h.shape
  E = r.shape[0]
  workers = _NC * _NS
  eidx = edge_index.reshape(2, workers, _NBLK, _BLKCH, _CHUNK)
  psum, pcnt = _sc_segment_sum(N, E, D)(eidx, h, r)
  psum = psum.reshape(_NC, N, D)
  pcnt = pcnt.reshape(_NC, N // 1000, 1, 1000)
  return _tc_finish(N, D)(psum, pcnt, W, b.reshape(1, D))


# speedup vs baseline: 1.1513x; 1.0126x over previous
import functools

import jax
import jax.numpy as jnp
from jax import lax
from jax.experimental import pallas as pl
from jax.experimental.pallas import tpu as pltpu
from jax.experimental.pallas import tpu_sc as plsc

_NC = 2
_NS = 16
_LANES = 16
_CHUNK = 40
_NCHUNKS = 250
_BLKCH = 10
_NBLK = _NCHUNKS // _BLKCH
_SLABRING = 3
_EDGES_PER_W = _CHUNK * _NCHUNKS


def _sc_segment_sum(N, E, D):
  workers = _NC * _NS
  assert E == workers * _EDGES_PER_W
  slice_rows = 640
  last_rows = N - (_NS - 1) * slice_rows
  mesh = plsc.VectorSubcoreMesh(core_axis_name="c", subcore_axis_name="s")

  @functools.partial(
      pl.kernel,
      out_type=[
          jax.ShapeDtypeStruct((_NC * N, D), jnp.float32),
          jax.ShapeDtypeStruct((_NC * N,), jnp.float32),
      ],
      mesh=mesh,
      scratch_types=[
          pltpu.VMEM((_SLABRING, 2, _BLKCH, _CHUNK), jnp.int32),
          pltpu.VMEM((_CHUNK, D), jnp.float32),
          pltpu.VMEM((_CHUNK, D), jnp.float32),
          pltpu.VMEM((_CHUNK, D), jnp.float32),
          pltpu.VMEM((_CHUNK, D), jnp.float32),
          pltpu.VMEM((_CHUNK, D), jnp.float32),
          pltpu.VMEM((_CHUNK, D), jnp.float32),
          pltpu.VMEM((_CHUNK,), jnp.float32),
          pltpu.VMEM((640,), jnp.float32),
          pltpu.VMEM_SHARED((N, D), jnp.float32),
          pltpu.VMEM_SHARED((N,), jnp.float32),
          pltpu.SemaphoreType.DMA,
          pltpu.SemaphoreType.DMA,
          pltpu.SemaphoreType.DMA,
          pltpu.SemaphoreType.DMA,
          pltpu.SemaphoreType.DMA,
          pltpu.SemaphoreType.DMA,
          pltpu.SemaphoreType.DMA,
          pltpu.SemaphoreType.DMA,
          pltpu.SemaphoreType.DMA,
      ],
  )
  def k(eidx_hbm, h_hbm, r_hbm, psum_hbm, pcnt_hbm,
        idxr, h0, h1, r0, r1, m0, m1, ones_v, cnt_stage,
        accum_sh, cnt_sh, gs0, gs1, ss0, ss1, cs0, cs1, is0, is1, is2):
    c = lax.axis_index("c")
    s = lax.axis_index("s")
    w = s * _NC + c
    hrow = (h0, h1)
    rrow = (r0, r1)
    mrow = (m0, m1)
    gsem = (gs0, gs1)
    ssem = (ss0, ss1)
    csem = (cs0, cs1)
    isem = (is0, is1, is2)
    ebase = pl.multiple_of(w * _EDGES_PER_W, 8)

    zero16 = jnp.zeros((_LANES,), jnp.float32)
    one16 = jnp.ones((_LANES,), jnp.float32)

    def zrow(j, carry):
      for t in range(D // _LANES):
        m0[j, pl.ds(t * _LANES, _LANES)] = zero16
      return carry
    lax.fori_loop(0, _CHUNK, zrow, 0)
    for t in range(640 // _LANES):
      cnt_stage[pl.ds(t * _LANES, _LANES)] = zero16
    for t in range(_CHUNK // _LANES):
      ones_v[pl.ds(t * _LANES, _LANES)] = one16
    ones_v[pl.ds(_CHUNK - _LANES, _LANES)] = one16

    base_row = pl.multiple_of(s * slice_rows, 8)

    def _zero_rows(nrows):
      pltpu.async_copy(cnt_stage.at[pl.ds(0, nrows)],
                       cnt_sh.at[pl.ds(base_row, nrows)], gs0)
      for q in range(nrows // _CHUNK):
        ro = pl.multiple_of(base_row + q * _CHUNK, 8)
        pltpu.async_copy(m0, accum_sh.at[pl.ds(ro, _CHUNK), :], gs0)
      pltpu.make_async_copy(cnt_stage.at[pl.ds(0, nrows)],
                            cnt_sh.at[pl.ds(base_row, nrows)], gs0).wait()
      for q in range(nrows // _CHUNK):
        ro = pl.multiple_of(base_row + q * _CHUNK, 8)
        pltpu.make_async_copy(m0, accum_sh.at[pl.ds(ro, _CHUNK), :],
                              gs0).wait()

    @pl.when(s < _NS - 1)
    def _():
      _zero_rows(slice_rows)

    @pl.when(s == _NS - 1)
    def _():
      _zero_rows(last_rows)

    plsc.subcore_barrier()

    def _fetch(j, b, fslot, frow):
      e0 = pl.multiple_of(ebase + j * _CHUNK, 8)
      pltpu.async_copy(h_hbm.at[idxr.at[fslot, 0, frow]], hrow[b], gsem[b])
      pltpu.async_copy(r_hbm.at[pl.ds(e0, _CHUNK), :], rrow[b], gsem[b])

    def _wait_fetch(b):
      pltpu.make_async_copy(h_hbm.at[idxr.at[0, 0, 0]], hrow[b],
                            gsem[b]).wait()
      pltpu.make_async_copy(r_hbm.at[pl.ds(0, _CHUNK), :], rrow[b],
                            gsem[b]).wait()

    def _wait_scatter(b):
      pltpu.make_async_copy(mrow[b], accum_sh.at[idxr.at[0, 1, 0]],
                            ssem[b]).wait()
      pltpu.make_async_copy(ones_v, cnt_sh.at[idxr.at[0, 1, 0]],
                            csem[b]).wait()

    def _stage_slab(blk, slot):
      pltpu.async_copy(eidx_hbm.at[0, w, blk], idxr.at[slot, 0], isem[slot])
      pltpu.async_copy(eidx_hbm.at[1, w, blk], idxr.at[slot, 1], isem[slot])

    def _wait_slab(slot):
      pltpu.make_async_copy(eidx_hbm.at[0, w, 0], idxr.at[0, 0],
                            isem[slot]).wait()
      pltpu.make_async_copy(eidx_hbm.at[1, w, 0], idxr.at[0, 1],
                            isem[slot]).wait()

    def _chunk(j, b, slot, row, fslot, frow):
      _wait_fetch(b)
      @pl.when(j >= 2)
      def _():
        _wait_scatter(b)

      def addrow(p2, carry2):
        j2 = p2 * 2
        for u in range(2):
          for t in range(D // _LANES):
            sl = pl.ds(t * _LANES, _LANES)
            mrow[b][j2 + u, sl] = hrow[b][j2 + u, sl] + rrow[b][j2 + u, sl]
        return carry2
      lax.fori_loop(0, _CHUNK // 2, addrow, 0)

      pltpu.async_copy(mrow[b], accum_sh.at[idxr.at[slot, 1, row]], ssem[b],
                       add=True)
      pltpu.async_copy(ones_v, cnt_sh.at[idxr.at[slot, 1, row]], csem[b],
                       add=True)

      @pl.when(j + 2 < _NCHUNKS)
      def _():
        _fetch(j + 2, b, fslot, frow)

    pltpu.sync_copy(eidx_hbm.at[0, w, 0], idxr.at[0, 0])
    pltpu.sync_copy(eidx_hbm.at[1, w, 0], idxr.at[0, 1])
    _fetch(0, 0, 0, 0)
    _fetch(1, 1, 0, 1)

    def _block(blk, slot, last):
      nslot = (slot + 1) % _SLABRING
      jbase = blk * _BLKCH
      if not last:
        _stage_slab(blk + 1, nslot)

      def pair_body(p, carry2):
        row = p * 2
        j = jbase + row
        _chunk(j, 0, slot, row, slot, row + 2)
        _chunk(j + 1, 1, slot, row + 1, slot, row + 3)
        return carry2
      lax.fori_loop(0, _BLKCH // 2 - 1, pair_body, 0)

      if not last:
        _wait_slab(nslot)
      _chunk(jbase + _BLKCH - 2, 0, slot, _BLKCH - 2, nslot, 0)
      _chunk(jbase + _BLKCH - 1, 1, slot, _BLKCH - 1, nslot, 1)

    def sblk_body(q, carry):
      blk = q * _SLABRING
      for o in range(_SLABRING):
        _block(blk + o, o, False)
      return carry
    lax.fori_loop(0, (_NBLK - 1) // _SLABRING, sblk_body, 0)
    _block(_NBLK - 1, (_NBLK - 1) % _SLABRING, True)

    _wait_scatter(0)
    _wait_scatter(1)

    plsc.subcore_barrier()

    def _writeback(nrows):
      ro = base_row
      out_ro = pl.multiple_of(c * N + base_row, 8)
      pltpu.async_copy(accum_sh.at[pl.ds(ro, nrows), :],
                       psum_hbm.at[pl.ds(out_ro, nrows), :], gs0)
      pltpu.sync_copy(cnt_sh.at[pl.ds(ro, nrows)],
                      cnt_stage.at[pl.ds(0, nrows)])
      pltpu.sync_copy(cnt_stage.at[pl.ds(0, nrows)],
                      pcnt_hbm.at[pl.ds(out_ro, nrows)])
      pltpu.make_async_copy(accum_sh.at[pl.ds(ro, nrows), :],
                            psum_hbm.at[pl.ds(out_ro, nrows), :], gs0).wait()

    @pl.when(s < _NS - 1)
    def _():
      _writeback(slice_rows)

    @pl.when(s == _NS - 1)
    def _():
      _writeback(last_rows)

  return k


_TCBLK = 2000


def _tc_finish(N, D):
  blk = _TCBLK
  def body(ps_ref, pc_ref, w_ref, b_ref, o_ref):
    ssum = ps_ref[0] + ps_ref[1]
    cnt = (pc_ref[0, 0, 0] + pc_ref[1, 0, 0]).reshape(blk, 1)
    feat = ssum / jnp.maximum(cnt, 1.0)
    o_ref[...] = jnp.dot(feat, w_ref[...],
                         preferred_element_type=jnp.float32) + b_ref[...]
  return pl.pallas_call(
      body,
      grid=(N // blk,),
      in_specs=[
          pl.BlockSpec((_NC, blk, D), lambda i: (0, i, 0)),
          pl.BlockSpec((_NC, 1, 1, blk), lambda i: (0, i, 0, 0)),
          pl.BlockSpec((D, D), lambda i: (0, 0)),
          pl.BlockSpec((1, D), lambda i: (0, 0)),
      ],
      out_specs=pl.BlockSpec((blk, D), lambda i: (i, 0)),
      out_shape=jax.ShapeDtypeStruct((N, D), jnp.float32),
  )


def kernel(h, r, edge_index, W, b):
  N, D = h.shape
  E = r.shape[0]
  workers = _NC * _NS
  eidx = edge_index.reshape(2, workers, _NBLK, _BLKCH, _CHUNK)
  psum, pcnt = _sc_segment_sum(N, E, D)(eidx, h, r)
  psum = psum.reshape(_NC, N, D)
  pcnt = pcnt.reshape(_NC, N // _TCBLK, 1, _TCBLK)
  return _tc_finish(N, D)(psum, pcnt, W, b.reshape(1, D))


# speedup vs baseline: 1.1551x; 1.0033x over previous
import functools

import jax
import jax.numpy as jnp
from jax import lax
from jax.experimental import pallas as pl
from jax.experimental.pallas import tpu as pltpu
from jax.experimental.pallas import tpu_sc as plsc

_NC = 2
_NS = 16
_LANES = 16
_CHUNK = 40
_NCHUNKS = 250
_BLKCH = 10
_NBLK = _NCHUNKS // _BLKCH
_SLABRING = 3
_EDGES_PER_W = _CHUNK * _NCHUNKS


def _sc_segment_sum(N, E, D):
  workers = _NC * _NS
  assert E == workers * _EDGES_PER_W
  slice_rows = 640
  last_rows = N - (_NS - 1) * slice_rows
  mesh = plsc.VectorSubcoreMesh(core_axis_name="c", subcore_axis_name="s")

  @functools.partial(
      pl.kernel,
      out_type=[
          jax.ShapeDtypeStruct((_NC * N, D), jnp.float32),
          jax.ShapeDtypeStruct((_NC * N,), jnp.float32),
      ],
      mesh=mesh,
      scratch_types=[
          pltpu.VMEM((_SLABRING, 2, _BLKCH, _CHUNK), jnp.int32),
          pltpu.VMEM((_CHUNK, D), jnp.float32),
          pltpu.VMEM((_CHUNK, D), jnp.float32),
          pltpu.VMEM((_CHUNK, D), jnp.float32),
          pltpu.VMEM((_CHUNK, D), jnp.float32),
          pltpu.VMEM((_CHUNK, D), jnp.float32),
          pltpu.VMEM((_CHUNK, D), jnp.float32),
          pltpu.VMEM((_CHUNK,), jnp.float32),
          pltpu.VMEM((640,), jnp.float32),
          pltpu.VMEM_SHARED((N, D), jnp.float32),
          pltpu.VMEM_SHARED((N,), jnp.float32),
          pltpu.SemaphoreType.DMA,
          pltpu.SemaphoreType.DMA,
          pltpu.SemaphoreType.DMA,
          pltpu.SemaphoreType.DMA,
          pltpu.SemaphoreType.DMA,
          pltpu.SemaphoreType.DMA,
          pltpu.SemaphoreType.DMA,
          pltpu.SemaphoreType.DMA,
          pltpu.SemaphoreType.DMA,
      ],
  )
  def k(eidx_hbm, h_hbm, r_hbm, psum_hbm, pcnt_hbm,
        idxr, h0, h1, r0, r1, m0, m1, ones_v, cnt_stage,
        accum_sh, cnt_sh, gs0, gs1, ss0, ss1, cs0, cs1, is0, is1, is2):
    c = lax.axis_index("c")
    s = lax.axis_index("s")
    w = s * _NC + c
    hrow = (h0, h1)
    rrow = (r0, r1)
    mrow = (m0, m1)
    gsem = (gs0, gs1)
    ssem = (ss0, ss1)
    csem = (cs0, cs1)
    isem = (is0, is1, is2)
    ebase = pl.multiple_of(w * _EDGES_PER_W, 8)

    zero16 = jnp.zeros((_LANES,), jnp.float32)
    one16 = jnp.ones((_LANES,), jnp.float32)

    def zrow(j, carry):
      for t in range(D // _LANES):
        m0[j, pl.ds(t * _LANES, _LANES)] = zero16
      return carry
    lax.fori_loop(0, _CHUNK, zrow, 0)
    for t in range(640 // _LANES):
      cnt_stage[pl.ds(t * _LANES, _LANES)] = zero16
    for t in range(_CHUNK // _LANES):
      ones_v[pl.ds(t * _LANES, _LANES)] = one16
    ones_v[pl.ds(_CHUNK - _LANES, _LANES)] = one16

    base_row = pl.multiple_of(s * slice_rows, 8)

    def _zero_rows(nrows):
      pltpu.async_copy(cnt_stage.at[pl.ds(0, nrows)],
                       cnt_sh.at[pl.ds(base_row, nrows)], gs0)
      for q in range(nrows // _CHUNK):
        ro = pl.multiple_of(base_row + q * _CHUNK, 8)
        pltpu.async_copy(m0, accum_sh.at[pl.ds(ro, _CHUNK), :], gs0)
      pltpu.make_async_copy(cnt_stage.at[pl.ds(0, nrows)],
                            cnt_sh.at[pl.ds(base_row, nrows)], gs0).wait()
      for q in range(nrows // _CHUNK):
        ro = pl.multiple_of(base_row + q * _CHUNK, 8)
        pltpu.make_async_copy(m0, accum_sh.at[pl.ds(ro, _CHUNK), :],
                              gs0).wait()

    @pl.when(s < _NS - 1)
    def _():
      _zero_rows(slice_rows)

    @pl.when(s == _NS - 1)
    def _():
      _zero_rows(last_rows)

    plsc.subcore_barrier()

    def _fetch(j, b, fslot, frow):
      e0 = pl.multiple_of(ebase + j * _CHUNK, 8)
      pltpu.async_copy(h_hbm.at[idxr.at[fslot, 0, frow]], hrow[b], gsem[b])
      pltpu.async_copy(r_hbm.at[pl.ds(e0, _CHUNK), :], rrow[b], gsem[b])

    def _wait_fetch(b):
      pltpu.make_async_copy(h_hbm.at[idxr.at[0, 0, 0]], hrow[b],
                            gsem[b]).wait()
      pltpu.make_async_copy(r_hbm.at[pl.ds(0, _CHUNK), :], rrow[b],
                            gsem[b]).wait()

    def _wait_scatter(b):
      pltpu.make_async_copy(mrow[b], accum_sh.at[idxr.at[0, 1, 0]],
                            ssem[b]).wait()
      pltpu.make_async_copy(ones_v, cnt_sh.at[idxr.at[0, 1, 0]],
                            csem[b]).wait()

    def _stage_slab(blk, slot):
      pltpu.async_copy(eidx_hbm.at[0, w, blk], idxr.at[slot, 0], isem[slot])
      pltpu.async_copy(eidx_hbm.at[1, w, blk], idxr.at[slot, 1], isem[slot])

    def _wait_slab(slot):
      pltpu.make_async_copy(eidx_hbm.at[0, w, 0], idxr.at[0, 0],
                            isem[slot]).wait()
      pltpu.make_async_copy(eidx_hbm.at[1, w, 0], idxr.at[0, 1],
                            isem[slot]).wait()

    def _chunk(j, b, slot, row, fslot, frow):
      _wait_fetch(b)
      @pl.when(j >= 2)
      def _():
        _wait_scatter(b)

      def addrow(p2, carry2):
        j2 = p2 * 2
        for u in range(2):
          for t in range(D // _LANES):
            sl = pl.ds(t * _LANES, _LANES)
            mrow[b][j2 + u, sl] = hrow[b][j2 + u, sl] + rrow[b][j2 + u, sl]
        return carry2
      lax.fori_loop(0, _CHUNK // 2, addrow, 0)

      pltpu.async_copy(mrow[b], accum_sh.at[idxr.at[slot, 1, row]], ssem[b],
                       add=True)
      pltpu.async_copy(ones_v, cnt_sh.at[idxr.at[slot, 1, row]], csem[b],
                       add=True)

      @pl.when(j + 2 < _NCHUNKS)
      def _():
        _fetch(j + 2, b, fslot, frow)

    pltpu.sync_copy(eidx_hbm.at[0, w, 0], idxr.at[0, 0])
    pltpu.sync_copy(eidx_hbm.at[1, w, 0], idxr.at[0, 1])
    _fetch(0, 0, 0, 0)
    _fetch(1, 1, 0, 1)

    def _block(blk, slot, last):
      nslot = (slot + 1) % _SLABRING
      jbase = blk * _BLKCH
      if not last:
        _stage_slab(blk + 1, nslot)

      def pair_body(p, carry2):
        row = p * 2
        j = jbase + row
        _chunk(j, 0, slot, row, slot, row + 2)
        _chunk(j + 1, 1, slot, row + 1, slot, row + 3)
        return carry2
      lax.fori_loop(0, _BLKCH // 2 - 1, pair_body, 0)

      if not last:
        _wait_slab(nslot)
      _chunk(jbase + _BLKCH - 2, 0, slot, _BLKCH - 2, nslot, 0)
      _chunk(jbase + _BLKCH - 1, 1, slot, _BLKCH - 1, nslot, 1)

    def sblk_body(q, carry):
      blk = q * _SLABRING
      for o in range(_SLABRING):
        _block(blk + o, o, False)
      return carry
    lax.fori_loop(0, (_NBLK - 1) // _SLABRING, sblk_body, 0)
    _block(_NBLK - 1, (_NBLK - 1) % _SLABRING, True)

    _wait_scatter(0)
    _wait_scatter(1)

    plsc.subcore_barrier()

    def _writeback(nrows):
      ro = base_row
      out_ro = pl.multiple_of(c * N + base_row, 8)
      pltpu.async_copy(accum_sh.at[pl.ds(ro, nrows), :],
                       psum_hbm.at[pl.ds(out_ro, nrows), :], gs0)
      pltpu.sync_copy(cnt_sh.at[pl.ds(ro, nrows)],
                      cnt_stage.at[pl.ds(0, nrows)])
      pltpu.sync_copy(cnt_stage.at[pl.ds(0, nrows)],
                      pcnt_hbm.at[pl.ds(out_ro, nrows)])
      pltpu.make_async_copy(accum_sh.at[pl.ds(ro, nrows), :],
                            psum_hbm.at[pl.ds(out_ro, nrows), :], gs0).wait()

    @pl.when(s < _NS - 1)
    def _():
      _writeback(slice_rows)

    @pl.when(s == _NS - 1)
    def _():
      _writeback(last_rows)

  return k


_TCBLK = 10000


def _tc_finish(N, D):
  blk = _TCBLK
  def body(ps_ref, pc_ref, w_ref, b_ref, o_ref):
    ssum = ps_ref[0] + ps_ref[1]
    cnt = (pc_ref[0, 0, 0] + pc_ref[1, 0, 0]).reshape(blk, 1)
    feat = ssum / jnp.maximum(cnt, 1.0)
    o_ref[...] = jnp.dot(feat, w_ref[...],
                         preferred_element_type=jnp.float32) + b_ref[...]
  return pl.pallas_call(
      body,
      grid=(N // blk,),
      in_specs=[
          pl.BlockSpec((_NC, blk, D), lambda i: (0, i, 0)),
          pl.BlockSpec((_NC, 1, 1, blk), lambda i: (0, i, 0, 0)),
          pl.BlockSpec((D, D), lambda i: (0, 0)),
          pl.BlockSpec((1, D), lambda i: (0, 0)),
      ],
      out_specs=pl.BlockSpec((blk, D), lambda i: (i, 0)),
      out_shape=jax.ShapeDtypeStruct((N, D), jnp.float32),
  )


def kernel(h, r, edge_index, W, b):
  N, D = h.shape
  E = r.shape[0]
  workers = _NC * _NS
  eidx = edge_index.reshape(2, workers, _NBLK, _BLKCH, _CHUNK)
  psum, pcnt = _sc_segment_sum(N, E, D)(eidx, h, r)
  psum = psum.reshape(_NC, N, D)
  pcnt = pcnt.reshape(_NC, N // _TCBLK, 1, _TCBLK)
  return _tc_finish(N, D)(psum, pcnt, W, b.reshape(1, D))


# speedup vs baseline: 1.1597x; 1.0040x over previous
import functools

import jax
import jax.numpy as jnp
from jax import lax
from jax.experimental import pallas as pl
from jax.experimental.pallas import tpu as pltpu
from jax.experimental.pallas import tpu_sc as plsc

_NC = 2
_NS = 16
_LANES = 16
_CHUNK = 40
_NCHUNKS = 250
_BLKCH = 10
_NBLK = _NCHUNKS // _BLKCH
_SLABRING = 3
_EDGES_PER_W = _CHUNK * _NCHUNKS


def _sc_segment_sum(N, E, D):
  workers = _NC * _NS
  assert E == workers * _EDGES_PER_W
  slice_rows = 640
  last_rows = N - (_NS - 1) * slice_rows
  mesh = plsc.VectorSubcoreMesh(core_axis_name="c", subcore_axis_name="s")

  @functools.partial(
      pl.kernel,
      out_type=[
          jax.ShapeDtypeStruct((_NC * N, D), jnp.float32),
          jax.ShapeDtypeStruct((_NC * N,), jnp.float32),
      ],
      mesh=mesh,
      scratch_types=[
          pltpu.VMEM((_SLABRING, 2, _BLKCH, _CHUNK), jnp.int32),
          pltpu.VMEM((_CHUNK, D), jnp.float32),
          pltpu.VMEM((_CHUNK, D), jnp.float32),
          pltpu.VMEM((_CHUNK, D), jnp.float32),
          pltpu.VMEM((_CHUNK, D), jnp.float32),
          pltpu.VMEM((_CHUNK, D), jnp.float32),
          pltpu.VMEM((_CHUNK, D), jnp.float32),
          pltpu.VMEM((_CHUNK,), jnp.float32),
          pltpu.VMEM((640,), jnp.float32),
          pltpu.VMEM_SHARED((N, D), jnp.float32),
          pltpu.VMEM_SHARED((N,), jnp.float32),
          pltpu.SemaphoreType.DMA,
          pltpu.SemaphoreType.DMA,
          pltpu.SemaphoreType.DMA,
          pltpu.SemaphoreType.DMA,
          pltpu.SemaphoreType.DMA,
          pltpu.SemaphoreType.DMA,
          pltpu.SemaphoreType.DMA,
          pltpu.SemaphoreType.DMA,
          pltpu.SemaphoreType.DMA,
      ],
  )
  def k(eidx_hbm, h_hbm, r_hbm, psum_hbm, pcnt_hbm,
        idxr, h0, h1, r0, r1, m0, m1, ones_v, cnt_stage,
        accum_sh, cnt_sh, gs0, gs1, ss0, ss1, cs0, cs1, is0, is1, is2):
    c = lax.axis_index("c")
    s = lax.axis_index("s")
    w = s * _NC + c
    hrow = (h0, h1)
    rrow = (r0, r1)
    mrow = (m0, m1)
    gsem = (gs0, gs1)
    ssem = (ss0, ss1)
    csem = (cs0, cs1)
    isem = (is0, is1, is2)
    ebase = pl.multiple_of(w * _EDGES_PER_W, 8)

    zero16 = jnp.zeros((_LANES,), jnp.float32)
    one16 = jnp.ones((_LANES,), jnp.float32)

    def zrow(j, carry):
      for t in range(D // _LANES):
        m0[j, pl.ds(t * _LANES, _LANES)] = zero16
      return carry
    lax.fori_loop(0, _CHUNK, zrow, 0)
    for t in range(640 // _LANES):
      cnt_stage[pl.ds(t * _LANES, _LANES)] = zero16
    for t in range(_CHUNK // _LANES):
      ones_v[pl.ds(t * _LANES, _LANES)] = one16
    ones_v[pl.ds(_CHUNK - _LANES, _LANES)] = one16

    pltpu.sync_copy(eidx_hbm.at[0, w, 0], idxr.at[0, 0])
    pltpu.sync_copy(eidx_hbm.at[1, w, 0], idxr.at[0, 1])
    pltpu.async_copy(h_hbm.at[idxr.at[0, 0, 0]], h0, gs0)
    pltpu.async_copy(r_hbm.at[pl.ds(ebase, _CHUNK), :], r0, gs0)
    e1 = pl.multiple_of(ebase + _CHUNK, 8)
    pltpu.async_copy(h_hbm.at[idxr.at[0, 0, 1]], h1, gs1)
    pltpu.async_copy(r_hbm.at[pl.ds(e1, _CHUNK), :], r1, gs1)

    base_row = pl.multiple_of(s * slice_rows, 8)

    def _zero_rows(nrows):
      pltpu.async_copy(cnt_stage.at[pl.ds(0, nrows)],
                       cnt_sh.at[pl.ds(base_row, nrows)], ss0)
      for q in range(nrows // _CHUNK):
        ro = pl.multiple_of(base_row + q * _CHUNK, 8)
        pltpu.async_copy(m0, accum_sh.at[pl.ds(ro, _CHUNK), :], ss0)
      pltpu.make_async_copy(cnt_stage.at[pl.ds(0, nrows)],
                            cnt_sh.at[pl.ds(base_row, nrows)], ss0).wait()
      for q in range(nrows // _CHUNK):
        ro = pl.multiple_of(base_row + q * _CHUNK, 8)
        pltpu.make_async_copy(m0, accum_sh.at[pl.ds(ro, _CHUNK), :],
                              ss0).wait()

    @pl.when(s < _NS - 1)
    def _():
      _zero_rows(slice_rows)

    @pl.when(s == _NS - 1)
    def _():
      _zero_rows(last_rows)

    plsc.subcore_barrier()

    def _fetch(j, b, fslot, frow):
      e0 = pl.multiple_of(ebase + j * _CHUNK, 8)
      pltpu.async_copy(h_hbm.at[idxr.at[fslot, 0, frow]], hrow[b], gsem[b])
      pltpu.async_copy(r_hbm.at[pl.ds(e0, _CHUNK), :], rrow[b], gsem[b])

    def _wait_fetch(b):
      pltpu.make_async_copy(h_hbm.at[idxr.at[0, 0, 0]], hrow[b],
                            gsem[b]).wait()
      pltpu.make_async_copy(r_hbm.at[pl.ds(0, _CHUNK), :], rrow[b],
                            gsem[b]).wait()

    def _wait_scatter(b):
      pltpu.make_async_copy(mrow[b], accum_sh.at[idxr.at[0, 1, 0]],
                            ssem[b]).wait()
      pltpu.make_async_copy(ones_v, cnt_sh.at[idxr.at[0, 1, 0]],
                            csem[b]).wait()

    def _stage_slab(blk, slot):
      pltpu.async_copy(eidx_hbm.at[0, w, blk], idxr.at[slot, 0], isem[slot])
      pltpu.async_copy(eidx_hbm.at[1, w, blk], idxr.at[slot, 1], isem[slot])

    def _wait_slab(slot):
      pltpu.make_async_copy(eidx_hbm.at[0, w, 0], idxr.at[0, 0],
                            isem[slot]).wait()
      pltpu.make_async_copy(eidx_hbm.at[1, w, 0], idxr.at[0, 1],
                            isem[slot]).wait()

    def _chunk(j, b, slot, row, fslot, frow):
      _wait_fetch(b)
      @pl.when(j >= 2)
      def _():
        _wait_scatter(b)

      def addrow(p2, carry2):
        j2 = p2 * 2
        for u in range(2):
          for t in range(D // _LANES):
            sl = pl.ds(t * _LANES, _LANES)
            mrow[b][j2 + u, sl] = hrow[b][j2 + u, sl] + rrow[b][j2 + u, sl]
        return carry2
      lax.fori_loop(0, _CHUNK // 2, addrow, 0)

      pltpu.async_copy(mrow[b], accum_sh.at[idxr.at[slot, 1, row]], ssem[b],
                       add=True)
      pltpu.async_copy(ones_v, cnt_sh.at[idxr.at[slot, 1, row]], csem[b],
                       add=True)

      @pl.when(j + 2 < _NCHUNKS)
      def _():
        _fetch(j + 2, b, fslot, frow)

    def _block(blk, slot, last):
      nslot = (slot + 1) % _SLABRING
      jbase = blk * _BLKCH
      if not last:
        _stage_slab(blk + 1, nslot)

      def pair_body(p, carry2):
        row = p * 2
        j = jbase + row
        _chunk(j, 0, slot, row, slot, row + 2)
        _chunk(j + 1, 1, slot, row + 1, slot, row + 3)
        return carry2
      lax.fori_loop(0, _BLKCH // 2 - 1, pair_body, 0)

      if not last:
        _wait_slab(nslot)
      _chunk(jbase + _BLKCH - 2, 0, slot, _BLKCH - 2, nslot, 0)
      _chunk(jbase + _BLKCH - 1, 1, slot, _BLKCH - 1, nslot, 1)

    def sblk_body(q, carry):
      blk = q * _SLABRING
      for o in range(_SLABRING):
        _block(blk + o, o, False)
      return carry
    lax.fori_loop(0, (_NBLK - 1) // _SLABRING, sblk_body, 0)
    _block(_NBLK - 1, (_NBLK - 1) % _SLABRING, True)

    _wait_scatter(0)
    _wait_scatter(1)

    plsc.subcore_barrier()

    def _writeback(nrows):
      ro = base_row
      out_ro = pl.multiple_of(c * N + base_row, 8)
      pltpu.async_copy(accum_sh.at[pl.ds(ro, nrows), :],
                       psum_hbm.at[pl.ds(out_ro, nrows), :], gs0)
      pltpu.sync_copy(cnt_sh.at[pl.ds(ro, nrows)],
                      cnt_stage.at[pl.ds(0, nrows)])
      pltpu.sync_copy(cnt_stage.at[pl.ds(0, nrows)],
                      pcnt_hbm.at[pl.ds(out_ro, nrows)])
      pltpu.make_async_copy(accum_sh.at[pl.ds(ro, nrows), :],
                            psum_hbm.at[pl.ds(out_ro, nrows), :], gs0).wait()

    @pl.when(s < _NS - 1)
    def _():
      _writeback(slice_rows)

    @pl.when(s == _NS - 1)
    def _():
      _writeback(last_rows)

  return k


_TCBLK = 10000


def _tc_finish(N, D):
  blk = _TCBLK
  def body(ps_ref, pc_ref, w_ref, b_ref, o_ref):
    ssum = ps_ref[0] + ps_ref[1]
    cnt = (pc_ref[0, 0, 0] + pc_ref[1, 0, 0]).reshape(blk, 1)
    feat = ssum / jnp.maximum(cnt, 1.0)
    o_ref[...] = jnp.dot(feat, w_ref[...],
                         preferred_element_type=jnp.float32) + b_ref[...]
  return pl.pallas_call(
      body,
      grid=(N // blk,),
      in_specs=[
          pl.BlockSpec((_NC, blk, D), lambda i: (0, i, 0)),
          pl.BlockSpec((_NC, 1, 1, blk), lambda i: (0, i, 0, 0)),
          pl.BlockSpec((D, D), lambda i: (0, 0)),
          pl.BlockSpec((1, D), lambda i: (0, 0)),
      ],
      out_specs=pl.BlockSpec((blk, D), lambda i: (i, 0)),
      out_shape=jax.ShapeDtypeStruct((N, D), jnp.float32),
  )


def kernel(h, r, edge_index, W, b):
  N, D = h.shape
  E = r.shape[0]
  workers = _NC * _NS
  eidx = edge_index.reshape(2, workers, _NBLK, _BLKCH, _CHUNK)
  psum, pcnt = _sc_segment_sum(N, E, D)(eidx, h, r)
  psum = psum.reshape(_NC, N, D)
  pcnt = pcnt.reshape(_NC, N // _TCBLK, 1, _TCBLK)
  return _tc_finish(N, D)(psum, pcnt, W, b.reshape(1, D))
